# baseline (speedup 1.0000x reference)
import dataclasses
import functools

import numpy as np

import jax
import jax.numpy as jnp
from jax import lax
from jax.experimental import pallas as pl
from jax.experimental.pallas import tpu as pltpu
from jax.experimental.pallas import tpu_sc as plsc

_NALL = 120000
_NLOC = 100000
_NSEL = 32
_SEL0 = 16
_QBINS = 1024
_QSCALE = _QBINS / 20.0
_QI2 = (20.0 / _QBINS) ** 2
_B = 96
_NW = 32
_NBLK = -(-_NLOC // _B)
_PER_TILE = 2 * (-(-_NBLK // (2 * _NW)))
_LAST_S = _NLOC - _B

_SHARD = 7680
_SHARD_LAST = _NALL - _SHARD

_NLUT = 1312


def _bf16_bits(x):
    b = np.asarray(x, np.float32).view(np.uint32)
    return ((b + 0x7FFF + ((b >> 16) & 1)) >> 16).astype(np.uint32)


def _make_lut(rcut_sq):
    r2c = np.arange(_NLUT) * 128.0 + 63.5
    u = r2c * _QI2 / rcut_sq
    return np.where(
        u < 1.0, 0.5 + 0.5 * np.cos(np.pi * np.sqrt(np.minimum(u, 1.0))), 0.0
    ).astype(np.float32)


_LUTP = (
    (_bf16_bits(_make_lut(64.0)) << 16) | _bf16_bits(_make_lut(36.0))
).view(np.int32)

_CP = pltpu.CompilerParams()
if "needs_layout_passes" in pltpu.CompilerParams.__dataclass_fields__:
    _CP = dataclasses.replace(_CP, needs_layout_passes=False)
if "use_tc_tiling_on_sc" in pltpu.CompilerParams.__dataclass_fields__:
    _CP = dataclasses.replace(_CP, use_tc_tiling_on_sc=False)


@functools.partial(
    pl.kernel,
    compiler_params=_CP,
    out_type=(
        jax.ShapeDtypeStruct((_NLOC,), jnp.float32),
        jax.ShapeDtypeStruct((_NALL,), jnp.int32),
    ),
    mesh=plsc.VectorSubcoreMesh(core_axis_name="c", subcore_axis_name="s"),
    scratch_types=[
        pltpu.VMEM((_NALL,), jnp.int32),
        pltpu.VMEM((_B, 33), jnp.int32),
        pltpu.VMEM((_B, 33), jnp.int32),
        pltpu.VMEM((_B,), jnp.float32),
        pltpu.VMEM((_B,), jnp.float32),
        pltpu.VMEM((_NLUT + 320,), jnp.int32),
        pltpu.SemaphoreType.DMA,
        pltpu.SemaphoreType.DMA,
        pltpu.SemaphoreType.DMA,
        pltpu.SemaphoreType.DMA,
    ],
)
def _sc_energy(crd_hbm, at_hbm, nl_hbm, par_hbm, out_hbm, tbl_hbm,
               tbl_v, idxA, idxB, outA, outB, par_v,
               semA, semB, semOA, semOB):
    sid = lax.axis_index("s")
    wid = sid * 2 + lax.axis_index("c")
    iota = lax.iota(jnp.int32, 16)
    iota3 = iota * 3

    def issue_in(idx_v, sem, blk):
        s = jnp.minimum(blk * _B, _LAST_S)
        pltpu.async_copy(nl_hbm.at[pl.ds(s, _B), :],
                         idx_v.at[:, pl.ds(0, _NSEL)], sem)

    def wait_in(idx_v, sem):
        pltpu.make_async_copy(nl_hbm.at[pl.ds(0, _B), :],
                              idx_v.at[:, pl.ds(0, _NSEL)], sem).wait()

    issue_in(idxA, semA, jnp.int32(wid))

    shard_s = jnp.minimum(sid * _SHARD, _SHARD_LAST)
    pltpu.async_copy(crd_hbm.at[pl.ds(shard_s * 3, _SHARD * 3)],
                     tbl_v.at[pl.ds(0, _SHARD * 3)], semOA)
    pltpu.async_copy(at_hbm.at[pl.ds(shard_s, _SHARD)],
                     tbl_v.at[pl.ds(_SHARD * 3, _SHARD)], semOA)
    pltpu.sync_copy(par_hbm, par_v)
    pltpu.make_async_copy(crd_hbm.at[pl.ds(0, _SHARD * 3)],
                          tbl_v.at[pl.ds(0, _SHARD * 3)], semOA).wait()
    pltpu.make_async_copy(at_hbm.at[pl.ds(0, _SHARD)],
                          tbl_v.at[pl.ds(_SHARD * 3, _SHARD)], semOA).wait()

    @pl.loop(0, _SHARD // 16)
    def _pgroups(g):
        b3 = iota3 + g * 48
        x = plsc.bitcast(plsc.load_gather(tbl_v, [b3]), jnp.float32)
        y = plsc.bitcast(plsc.load_gather(tbl_v, [b3 + 1]), jnp.float32)
        z = plsc.bitcast(plsc.load_gather(tbl_v, [b3 + 2]), jnp.float32)
        qx = jnp.minimum((x * jnp.float32(_QSCALE)).astype(jnp.int32), 1023)
        qy = jnp.minimum((y * jnp.float32(_QSCALE)).astype(jnp.int32), 1023)
        qz = jnp.minimum((z * jnp.float32(_QSCALE)).astype(jnp.int32), 1023)
        t = tbl_v[pl.ds(_SHARD * 3 + g * 16, 16)]
        w = qx | (qy << 10) | (qz << 20) | (t << 30)
        tbl_v[pl.ds(_SHARD * 4 + g * 16, 16)] = w

    pltpu.sync_copy(tbl_v.at[pl.ds(_SHARD * 4, _SHARD)],
                    tbl_hbm.at[pl.ds(shard_s, _SHARD)])
    plsc.subcore_barrier()

    pltpu.sync_copy(tbl_hbm, tbl_v)

    def compute(idx_v, out_v, blk):
        s = jnp.minimum(blk * _B, _LAST_S)

        @pl.loop(0, _B // 16)
        def _groups(ag):
            a0 = ag * 16
            wc = tbl_v[pl.ds(s + a0, 16)]
            cx = wc & 1023
            cy = (wc >> 10) & 1023
            cz = (wc >> 20) & 1023
            ti = (wc >> 30) & 3
            ti64 = (ti << 6) + iota + _NLUT
            rows = iota + a0
            acc0 = jnp.zeros((16,), jnp.float32)
            acc1 = jnp.zeros((16,), jnp.float32)
            for j in range(_NSEL):
                n = plsc.load_gather(idx_v, [rows, jnp.full((16,), j, jnp.int32)])
                w = plsc.load_gather(tbl_v, [n])
                dx = (w & 1023) - cx
                dy = ((w >> 10) & 1023) - cy
                dz = ((w >> 20) & 1023) - cz
                tj = (w >> 30) & 3
                r2 = dx * dx + dy * dy + dz * dz
                li = jnp.minimum(r2 >> 7, _NLUT - 1)
                ci = ti64 + (tj << 4)
                wp = plsc.load_gather(par_v, [ci])
                wl = plsc.load_gather(par_v, [li])
                hi = jnp.int32(-65536)
                c1 = plsc.bitcast(wp & hi, jnp.float32)
                sw1 = plsc.bitcast(wl & hi, jnp.float32)
                acc1 = acc1 + c1 * sw1
                if j < _SEL0:
                    c0 = plsc.bitcast(wp << 16, jnp.float32)
                    sw0 = plsc.bitcast(wl << 16, jnp.float32)
                    acc0 = acc0 + c0 * sw0
            bsw = plsc.load_gather(par_v, [(ti << 4) + iota + (_NLUT + 256)])
            e = (acc0 + acc1) * jnp.float32(0.5) + plsc.bitcast(
                bsw, jnp.float32)
            out_v[pl.ds(a0, 16)] = e

        return s

    @pl.loop(0, _PER_TILE // 2)
    def _pairs(p):
        blk0 = p * 2 * _NW + wid
        blk1 = blk0 + _NW
        blk2 = blk1 + _NW
        wait_in(idxA, semA)
        issue_in(idxB, semB, blk1)

        @pl.when(p > 0)
        def _():
            pltpu.make_async_copy(outA, out_hbm.at[pl.ds(0, _B)], semOA).wait()

        sA = compute(idxA, outA, blk0)
        pltpu.async_copy(outA, out_hbm.at[pl.ds(sA, _B)], semOA)
        wait_in(idxB, semB)

        @pl.when(p < _PER_TILE // 2 - 1)
        def _():
            issue_in(idxA, semA, blk2)

        @pl.when(p > 0)
        def _():
            pltpu.make_async_copy(outB, out_hbm.at[pl.ds(0, _B)], semOB).wait()

        sB = compute(idxB, outB, blk1)
        pltpu.async_copy(outB, out_hbm.at[pl.ds(sB, _B)], semOB)

    pltpu.make_async_copy(outA, out_hbm.at[pl.ds(0, _B)], semOA).wait()
    pltpu.make_async_copy(outB, out_hbm.at[pl.ds(0, _B)], semOB).wait()


def kernel(extended_coord, extended_atype, nlist, pair_coef0, pair_coef1,
           bias0, bias1):
    nframes = extended_coord.shape[0]
    crd = lax.bitcast_convert_type(
        extended_coord.reshape(_NALL * 3), jnp.int32)
    atype = extended_atype.reshape(_NALL).astype(jnp.int32)
    nl = nlist.reshape(_NLOC, _NSEL).astype(jnp.int32)
    p0b = lax.bitcast_convert_type(
        pair_coef0.reshape(16).astype(jnp.bfloat16), jnp.uint16
    ).astype(jnp.uint32)
    p1b = lax.bitcast_convert_type(
        pair_coef1.reshape(16).astype(jnp.bfloat16), jnp.uint16
    ).astype(jnp.uint32)
    pcp = jnp.repeat(
        lax.bitcast_convert_type((p1b << 16) | p0b, jnp.int32), 16)
    bs = jnp.repeat((bias0 + bias1) * jnp.float32(0.5), 16)
    params = jnp.concatenate(
        [jnp.asarray(_LUTP), pcp, lax.bitcast_convert_type(bs, jnp.int32)])
    energy, _ = _sc_energy(crd, atype, nl, params)
    return energy.reshape(nframes, _NLOC)


# speedup vs baseline: 1.0607x; 1.0607x over previous
import dataclasses
import functools

import numpy as np

import jax
import jax.numpy as jnp
from jax import lax
from jax.experimental import pallas as pl
from jax.experimental.pallas import tpu as pltpu
from jax.experimental.pallas import tpu_sc as plsc

_NALL = 120000
_NLOC = 100000
_NSEL = 32
_SEL0 = 16
_QBINS = 1024
_QSCALE = _QBINS / 20.0
_QI2 = (20.0 / _QBINS) ** 2
_B = 96
_NW = 32
_NBLK = -(-_NLOC // _B)
_PER_TILE = 2 * (-(-_NBLK // (2 * _NW)))
_LAST_S = _NLOC - _B

_SHARD = 7680
_SHARD_LAST = _NALL - _SHARD

_NLUT = 1312


def _bf16_bits(x):
    b = np.asarray(x, np.float32).view(np.uint32)
    return ((b + 0x7FFF + ((b >> 16) & 1)) >> 16).astype(np.uint32)


def _make_lut(rcut_sq):
    r2c = np.arange(_NLUT) * 128.0 + 63.5
    u = r2c * _QI2 / rcut_sq
    return np.where(
        u < 1.0, 0.5 + 0.5 * np.cos(np.pi * np.sqrt(np.minimum(u, 1.0))), 0.0
    ).astype(np.float32)


_LUTP = (
    (_bf16_bits(_make_lut(64.0)) << 16) | _bf16_bits(_make_lut(36.0))
).view(np.int32)

_CP = pltpu.CompilerParams()
if "needs_layout_passes" in pltpu.CompilerParams.__dataclass_fields__:
    _CP = dataclasses.replace(_CP, needs_layout_passes=False)
if "use_tc_tiling_on_sc" in pltpu.CompilerParams.__dataclass_fields__:
    _CP = dataclasses.replace(_CP, use_tc_tiling_on_sc=False)


@functools.partial(
    pl.kernel,
    compiler_params=_CP,
    out_type=jax.ShapeDtypeStruct((_NALL,), jnp.int32),
    mesh=plsc.VectorSubcoreMesh(core_axis_name="c", subcore_axis_name="s"),
    scratch_types=[
        pltpu.VMEM((_SHARD * 5,), jnp.int32),
        pltpu.SemaphoreType.DMA,
    ],
)
def _sc_pack(crd_hbm, at_hbm, tbl_hbm, st_v, sem):
    sid = lax.axis_index("s")
    iota = lax.iota(jnp.int32, 16)
    iota3 = iota * 3
    shard_s = jnp.minimum(sid * _SHARD, _SHARD_LAST)
    pltpu.async_copy(crd_hbm.at[pl.ds(shard_s * 3, _SHARD * 3)],
                     st_v.at[pl.ds(0, _SHARD * 3)], sem)
    pltpu.async_copy(at_hbm.at[pl.ds(shard_s, _SHARD)],
                     st_v.at[pl.ds(_SHARD * 3, _SHARD)], sem)
    pltpu.make_async_copy(crd_hbm.at[pl.ds(0, _SHARD * 3)],
                          st_v.at[pl.ds(0, _SHARD * 3)], sem).wait()
    pltpu.make_async_copy(at_hbm.at[pl.ds(0, _SHARD)],
                          st_v.at[pl.ds(_SHARD * 3, _SHARD)], sem).wait()

    @pl.loop(0, _SHARD // 16)
    def _pgroups(g):
        b3 = iota3 + g * 48
        x = plsc.bitcast(plsc.load_gather(st_v, [b3]), jnp.float32)
        y = plsc.bitcast(plsc.load_gather(st_v, [b3 + 1]), jnp.float32)
        z = plsc.bitcast(plsc.load_gather(st_v, [b3 + 2]), jnp.float32)
        qx = jnp.minimum((x * jnp.float32(_QSCALE)).astype(jnp.int32), 1023)
        qy = jnp.minimum((y * jnp.float32(_QSCALE)).astype(jnp.int32), 1023)
        qz = jnp.minimum((z * jnp.float32(_QSCALE)).astype(jnp.int32), 1023)
        t = st_v[pl.ds(_SHARD * 3 + g * 16, 16)]
        w = qx | (qy << 10) | (qz << 20) | (t << 30)
        st_v[pl.ds(_SHARD * 4 + g * 16, 16)] = w

    pltpu.sync_copy(st_v.at[pl.ds(_SHARD * 4, _SHARD)],
                    tbl_hbm.at[pl.ds(shard_s, _SHARD)])


@functools.partial(
    pl.kernel,
    compiler_params=_CP,
    out_type=jax.ShapeDtypeStruct((_NLOC,), jnp.float32),
    mesh=plsc.VectorSubcoreMesh(core_axis_name="c", subcore_axis_name="s"),
    scratch_types=[
        pltpu.VMEM((_NALL,), jnp.int32),
        pltpu.VMEM((_B, 33), jnp.int32),
        pltpu.VMEM((_B, 33), jnp.int32),
        pltpu.VMEM((_B,), jnp.float32),
        pltpu.VMEM((_B,), jnp.float32),
        pltpu.VMEM((_NLUT + 320,), jnp.int32),
        pltpu.SemaphoreType.DMA,
        pltpu.SemaphoreType.DMA,
        pltpu.SemaphoreType.DMA,
        pltpu.SemaphoreType.DMA,
    ],
)
def _sc_energy(nl_hbm, par_hbm, tbl_hbm, out_hbm,
               tbl_v, idxA, idxB, outA, outB, par_v,
               semA, semB, semOA, semOB):
    sid = lax.axis_index("s")
    wid = sid * 2 + lax.axis_index("c")
    iota = lax.iota(jnp.int32, 16)

    def issue_in(idx_v, sem, blk):
        s = jnp.minimum(blk * _B, _LAST_S)
        pltpu.async_copy(nl_hbm.at[pl.ds(s, _B), :],
                         idx_v.at[:, pl.ds(0, _NSEL)], sem)

    def wait_in(idx_v, sem):
        pltpu.make_async_copy(nl_hbm.at[pl.ds(0, _B), :],
                              idx_v.at[:, pl.ds(0, _NSEL)], sem).wait()

    issue_in(idxA, semA, jnp.int32(wid))
    pltpu.sync_copy(par_hbm, par_v)
    pltpu.sync_copy(tbl_hbm, tbl_v)

    def compute(idx_v, out_v, blk):
        s = jnp.minimum(blk * _B, _LAST_S)

        @pl.loop(0, _B // 16)
        def _groups(ag):
            a0 = ag * 16
            wc = tbl_v[pl.ds(s + a0, 16)]
            cx = wc & 1023
            cy = (wc >> 10) & 1023
            cz = (wc >> 20) & 1023
            ti = (wc >> 30) & 3
            ti64 = (ti << 6) + iota + _NLUT
            rows = iota + a0
            acc0 = jnp.zeros((16,), jnp.float32)
            acc1 = jnp.zeros((16,), jnp.float32)
            for j in range(_NSEL):
                n = plsc.load_gather(idx_v, [rows, jnp.full((16,), j, jnp.int32)])
                w = plsc.load_gather(tbl_v, [n])
                dx = (w & 1023) - cx
                dy = ((w >> 10) & 1023) - cy
                dz = ((w >> 20) & 1023) - cz
                tj = (w >> 30) & 3
                r2 = dx * dx + dy * dy + dz * dz
                li = jnp.minimum(r2 >> 7, _NLUT - 1)
                ci = ti64 + (tj << 4)
                wp = plsc.load_gather(par_v, [ci])
                wl = plsc.load_gather(par_v, [li])
                hi = jnp.int32(-65536)
                c1 = plsc.bitcast(wp & hi, jnp.float32)
                sw1 = plsc.bitcast(wl & hi, jnp.float32)
                acc1 = acc1 + c1 * sw1
                if j < _SEL0:
                    c0 = plsc.bitcast(wp << 16, jnp.float32)
                    sw0 = plsc.bitcast(wl << 16, jnp.float32)
                    acc0 = acc0 + c0 * sw0
            bsw = plsc.load_gather(par_v, [(ti << 4) + iota + (_NLUT + 256)])
            e = (acc0 + acc1) * jnp.float32(0.5) + plsc.bitcast(
                bsw, jnp.float32)
            out_v[pl.ds(a0, 16)] = e

        return s

    @pl.loop(0, _PER_TILE // 2)
    def _pairs(p):
        blk0 = p * 2 * _NW + wid
        blk1 = blk0 + _NW
        blk2 = blk1 + _NW
        wait_in(idxA, semA)
        issue_in(idxB, semB, blk1)

        @pl.when(p > 0)
        def _():
            pltpu.make_async_copy(outA, out_hbm.at[pl.ds(0, _B)], semOA).wait()

        sA = compute(idxA, outA, blk0)
        pltpu.async_copy(outA, out_hbm.at[pl.ds(sA, _B)], semOA)
        wait_in(idxB, semB)

        @pl.when(p < _PER_TILE // 2 - 1)
        def _():
            issue_in(idxA, semA, blk2)

        @pl.when(p > 0)
        def _():
            pltpu.make_async_copy(outB, out_hbm.at[pl.ds(0, _B)], semOB).wait()

        sB = compute(idxB, outB, blk1)
        pltpu.async_copy(outB, out_hbm.at[pl.ds(sB, _B)], semOB)

    pltpu.make_async_copy(outA, out_hbm.at[pl.ds(0, _B)], semOA).wait()
    pltpu.make_async_copy(outB, out_hbm.at[pl.ds(0, _B)], semOB).wait()


def kernel(extended_coord, extended_atype, nlist, pair_coef0, pair_coef1,
           bias0, bias1):
    nframes = extended_coord.shape[0]
    crd = lax.bitcast_convert_type(
        extended_coord.reshape(_NALL * 3), jnp.int32)
    atype = extended_atype.reshape(_NALL).astype(jnp.int32)
    nl = nlist.reshape(_NLOC, _NSEL).astype(jnp.int32)
    p0b = lax.bitcast_convert_type(
        pair_coef0.reshape(16).astype(jnp.bfloat16), jnp.uint16
    ).astype(jnp.uint32)
    p1b = lax.bitcast_convert_type(
        pair_coef1.reshape(16).astype(jnp.bfloat16), jnp.uint16
    ).astype(jnp.uint32)
    pcp = jnp.repeat(
        lax.bitcast_convert_type((p1b << 16) | p0b, jnp.int32), 16)
    bs = jnp.repeat((bias0 + bias1) * jnp.float32(0.5), 16)
    params = jnp.concatenate(
        [jnp.asarray(_LUTP), pcp, lax.bitcast_convert_type(bs, jnp.int32)])
    tbl = _sc_pack(crd, atype)
    energy = _sc_energy(nl, params, tbl)
    return energy.reshape(nframes, _NLOC)
